# baseline (speedup 1.0000x reference)
import functools

import jax
import jax.numpy as jnp
from jax.experimental import pallas as pl
from jax.experimental.pallas import tpu as pltpu


def _stage(v, idx, col, j, desc):
    bit = (col & j) != 0
    pv = jnp.where(bit, jnp.roll(v, j, axis=-1), jnp.roll(v, -j, axis=-1))
    pi = jnp.where(bit, jnp.roll(idx, j, axis=-1), jnp.roll(idx, -j, axis=-1))
    keep_first = desc == (~bit)
    self_wins = (v > pv) | ((v == pv) & (idx < pi))
    sel = self_wins ^ keep_first
    return jnp.where(sel, pv, v), jnp.where(sel, pi, idx)


def _topk_sort(v, idx, w, k):
    col = jax.lax.broadcasted_iota(jnp.int32, v.shape, v.ndim - 1)
    kk = 2
    while kk <= k:
        desc = (col & kk) == 0
        j = kk // 2
        while j >= 1:
            v, idx = _stage(v, idx, col, j, desc)
            j //= 2
        kk *= 2
    c = w // k
    while c > 1:
        npairs = c // 2
        head_w = 2 * npairs * k
        hv, hi = v[..., :head_w], idx[..., :head_w]
        hcol = jax.lax.broadcasted_iota(jnp.int32, hv.shape, hv.ndim - 1)
        hv, hi = _stage(hv, hi, hcol, k, hcol >= 0)
        keep_v = [hv[..., p * 2 * k : p * 2 * k + k] for p in range(npairs)]
        keep_i = [hi[..., p * 2 * k : p * 2 * k + k] for p in range(npairs)]
        if c % 2:
            keep_v.append(v[..., (c - 1) * k :])
            keep_i.append(idx[..., (c - 1) * k :])
        v = keep_v[0] if len(keep_v) == 1 else jnp.concatenate(keep_v, axis=-1)
        idx = keep_i[0] if len(keep_i) == 1 else jnp.concatenate(keep_i, axis=-1)
        c = (c + 1) // 2
        col = jax.lax.broadcasted_iota(jnp.int32, v.shape, v.ndim - 1)
        desc = (col & k) == 0
        j = k // 2
        while j >= 1:
            v, idx = _stage(v, idx, col, j, desc)
            j //= 2
    return v, idx


def _class_body(x_ref, mask_ref, idx_ref, *, rows_per_block, q0, w, seq_k, k):
    i = pl.program_id(1)
    x = x_ref[0]
    col = jax.lax.broadcasted_iota(jnp.int32, x.shape, 1)
    q = q0 + i * rows_per_block + jax.lax.broadcasted_iota(jnp.int32, x.shape, 0)
    vm = jnp.where(col > q, jnp.float32(-1e9), x)

    if w == k:
        _, topi = _topk_sort(vm, col, w, k)
        idx_ref[0] = topi
        colf = jax.lax.broadcasted_iota(
            jnp.int32, (rows_per_block, seq_k), 1
        )
        mask_ref[0] = (colf < k).astype(jnp.int32)
        return

    topv, topi = _topk_sort(vm, col, w, k)
    idx_ref[0] = topi
    thr = topv[:, k - 1 : k]
    tie_max = jnp.max(
        jnp.where(topv == thr, topi, jnp.int32(-1)), axis=1, keepdims=True
    )
    sel = (vm > thr) | ((vm == thr) & (col <= tie_max))
    mask_ref[0, :, :w] = sel.astype(jnp.int32)
    if w < seq_k:
        mask_ref[0, :, w:] = jnp.zeros(
            (rows_per_block, seq_k - w), jnp.int32
        )


def _run_class(x, q0, q1, w, k, rows_per_block, interpret=False):
    B, Sq, Sk = x.shape
    nrows = q1 - q0
    body = functools.partial(
        _class_body, rows_per_block=rows_per_block, q0=q0, w=w, seq_k=Sk, k=k
    )
    return pl.pallas_call(
        body,
        grid=(B, nrows // rows_per_block),
        in_specs=[
            pl.BlockSpec(
                (1, rows_per_block, w),
                lambda b, i, q0=q0, R=rows_per_block: (b, q0 // R + i, 0),
            )
        ],
        out_specs=[
            pl.BlockSpec((1, rows_per_block, Sk), lambda b, i: (b, i, 0)),
            pl.BlockSpec((1, rows_per_block, k), lambda b, i: (b, i, 0)),
        ],
        out_shape=[
            jax.ShapeDtypeStruct((B, nrows, Sk), jnp.int32),
            jax.ShapeDtypeStruct((B, nrows, k), jnp.int32),
        ],
        compiler_params=None
        if interpret
        else pltpu.CompilerParams(
            dimension_semantics=("parallel", "parallel")
        ),
        interpret=interpret,
    )(x)


def _run_topk(x, k, rows_per_block, interpret=False):
    B, Sq, Sk = x.shape
    masks, idxs = [], []
    for m in range(Sq // k):
        q0, q1 = m * k, (m + 1) * k
        w = min(q1, Sk)
        mm, ti = _run_class(x, q0, q1, w, k, rows_per_block, interpret)
        masks.append(mm)
        idxs.append(ti)
    mask = masks[0] if len(masks) == 1 else jnp.concatenate(masks, axis=1)
    topi = idxs[0] if len(idxs) == 1 else jnp.concatenate(idxs, axis=1)
    return mask, topi


def kernel(index_scores):
    B, Sq, Sk = index_scores.shape
    k = min(512, Sk)
    mask8, topi = _run_topk(index_scores, k, 8)
    mask = mask8 != 0
    sparsity = jnp.float32(1.0) - jnp.float32(k) / jnp.float32(Sk)
    return (mask, topi, sparsity)


# speedup vs baseline: 1.3973x; 1.3973x over previous
import functools

import jax
import jax.numpy as jnp
from jax.experimental import pallas as pl
from jax.experimental.pallas import tpu as pltpu


def _stage(v, idx, col, j, desc):
    bit = (col & j) != 0
    pv = jnp.where(bit, jnp.roll(v, j, axis=-1), jnp.roll(v, -j, axis=-1))
    pi = jnp.where(bit, jnp.roll(idx, j, axis=-1), jnp.roll(idx, -j, axis=-1))
    keep_first = desc ^ bit
    self_wins = (v > pv) | ((v == pv) & (idx < pi))
    sel = self_wins ^ keep_first
    return jnp.where(sel, pv, v), jnp.where(sel, pi, idx)


def _topk_sort(v, idx, w, k):
    col = jax.lax.broadcasted_iota(jnp.int32, v.shape, v.ndim - 1)
    kk = 2
    while kk <= k:
        desc = (col & kk) == 0
        j = kk // 2
        while j >= 1:
            v, idx = _stage(v, idx, col, j, desc)
            j //= 2
        kk *= 2
    c = w // k
    while c > 1:
        npairs = c // 2
        head_w = 2 * npairs * k
        hv, hi = v[..., :head_w], idx[..., :head_w]
        hcol = jax.lax.broadcasted_iota(jnp.int32, hv.shape, hv.ndim - 1)
        hv, hi = _stage(hv, hi, hcol, k, hcol >= 0)
        keep_v = [hv[..., p * 2 * k : p * 2 * k + k] for p in range(npairs)]
        keep_i = [hi[..., p * 2 * k : p * 2 * k + k] for p in range(npairs)]
        if c % 2:
            keep_v.append(v[..., (c - 1) * k :])
            keep_i.append(idx[..., (c - 1) * k :])
        v = keep_v[0] if len(keep_v) == 1 else jnp.concatenate(keep_v, axis=-1)
        idx = keep_i[0] if len(keep_i) == 1 else jnp.concatenate(keep_i, axis=-1)
        c = (c + 1) // 2
        col = jax.lax.broadcasted_iota(jnp.int32, v.shape, v.ndim - 1)
        desc = (col & k) == 0
        j = k // 2
        while j >= 1:
            v, idx = _stage(v, idx, col, j, desc)
            j //= 2
    return v, idx


def _class_body(x_ref, mask_ref, idx_ref, *, rows_per_block, q0, w, seq_k, k):
    i = pl.program_id(1)
    x = x_ref[0]
    col = jax.lax.broadcasted_iota(jnp.int32, x.shape, 1)
    q = q0 + i * rows_per_block + jax.lax.broadcasted_iota(jnp.int32, x.shape, 0)
    vm = jnp.where(col > q, jnp.float32(-1e9), x)

    if w == k:
        _, topi = _topk_sort(vm, col, w, k)
        idx_ref[0] = topi
        colf = jax.lax.broadcasted_iota(
            jnp.int32, (rows_per_block, seq_k), 1
        )
        mask_ref[0] = (colf < k).astype(jnp.int8)
        return

    topv, topi = _topk_sort(vm, col, w, k)
    idx_ref[0] = topi
    thr = topv[:, k - 1 : k]
    tie_max = jnp.max(
        jnp.where(topv == thr, topi, jnp.int32(-1)), axis=1, keepdims=True
    )
    sel = (vm > thr) | ((vm == thr) & (col <= tie_max))
    mask_ref[0, :, :w] = sel.astype(jnp.int8)
    if w < seq_k:
        mask_ref[0, :, w:] = jnp.zeros(
            (rows_per_block, seq_k - w), jnp.int8
        )


def _run_class(x, q0, q1, w, k, rows_per_block, interpret=False):
    B, Sq, Sk = x.shape
    nrows = q1 - q0
    body = functools.partial(
        _class_body, rows_per_block=rows_per_block, q0=q0, w=w, seq_k=Sk, k=k
    )
    return pl.pallas_call(
        body,
        grid=(B, nrows // rows_per_block),
        in_specs=[
            pl.BlockSpec(
                (1, rows_per_block, w),
                lambda b, i, q0=q0, R=rows_per_block: (b, q0 // R + i, 0),
            )
        ],
        out_specs=[
            pl.BlockSpec((1, rows_per_block, Sk), lambda b, i: (b, i, 0)),
            pl.BlockSpec((1, rows_per_block, k), lambda b, i: (b, i, 0)),
        ],
        out_shape=[
            jax.ShapeDtypeStruct((B, nrows, Sk), jnp.int8),
            jax.ShapeDtypeStruct((B, nrows, k), jnp.int32),
        ],
        compiler_params=None
        if interpret
        else pltpu.CompilerParams(
            dimension_semantics=("parallel", "parallel")
        ),
        interpret=interpret,
    )(x)


def _run_topk(x, k, rows_per_block, interpret=False):
    B, Sq, Sk = x.shape
    masks, idxs = [], []
    for m in range(Sq // k):
        q0, q1 = m * k, (m + 1) * k
        w = min(q1, Sk)
        mm, ti = _run_class(x, q0, q1, w, k, rows_per_block, interpret)
        masks.append(mm)
        idxs.append(ti)
    mask = masks[0] if len(masks) == 1 else jnp.concatenate(masks, axis=1)
    topi = idxs[0] if len(idxs) == 1 else jnp.concatenate(idxs, axis=1)
    return mask, topi


def kernel(index_scores):
    B, Sq, Sk = index_scores.shape
    k = min(512, Sk)
    mask8, topi = _run_topk(index_scores, k, 64)
    mask = mask8 != 0
    sparsity = jnp.float32(1.0) - jnp.float32(k) / jnp.float32(Sk)
    return (mask, topi, sparsity)


# speedup vs baseline: 1.7325x; 1.2399x over previous
import functools

import jax
import jax.numpy as jnp
from jax.experimental import pallas as pl
from jax.experimental.pallas import tpu as pltpu


def _stage(v, idx, col, j, desc):
    bit = (col & j) != 0
    pv = jnp.where(bit, jnp.roll(v, j, axis=-1), jnp.roll(v, -j, axis=-1))
    pi = jnp.where(bit, jnp.roll(idx, j, axis=-1), jnp.roll(idx, -j, axis=-1))
    keep_first = desc ^ bit
    self_wins = (v > pv) | ((v == pv) & (idx < pi))
    sel = self_wins ^ keep_first
    return jnp.where(sel, pv, v), jnp.where(sel, pi, idx)


def _topk_sort(v, idx, w, k):
    col = jax.lax.broadcasted_iota(jnp.int32, v.shape, v.ndim - 1)
    kk = 2
    while kk <= k:
        desc = (col & kk) == 0
        j = kk // 2
        while j >= 1:
            v, idx = _stage(v, idx, col, j, desc)
            j //= 2
        kk *= 2
    c = w // k
    while c > 1:
        npairs = c // 2
        head_w = 2 * npairs * k
        hv, hi = v[..., :head_w], idx[..., :head_w]
        hcol = jax.lax.broadcasted_iota(jnp.int32, hv.shape, hv.ndim - 1)
        hv, hi = _stage(hv, hi, hcol, k, hcol >= 0)
        keep_v = [hv[..., p * 2 * k : p * 2 * k + k] for p in range(npairs)]
        keep_i = [hi[..., p * 2 * k : p * 2 * k + k] for p in range(npairs)]
        if c % 2:
            keep_v.append(v[..., (c - 1) * k :])
            keep_i.append(idx[..., (c - 1) * k :])
        v = keep_v[0] if len(keep_v) == 1 else jnp.concatenate(keep_v, axis=-1)
        idx = keep_i[0] if len(keep_i) == 1 else jnp.concatenate(keep_i, axis=-1)
        c = (c + 1) // 2
        col = jax.lax.broadcasted_iota(jnp.int32, v.shape, v.ndim - 1)
        desc = (col & k) == 0
        j = k // 2
        while j >= 1:
            v, idx = _stage(v, idx, col, j, desc)
            j //= 2
    return v, idx


def _class_body(x_ref, mask_ref, idx_ref, *, rows_per_block, q0, w, seq_k, k):
    i = pl.program_id(1)
    x = x_ref[0]
    col = jax.lax.broadcasted_iota(jnp.int32, x.shape, 1)
    col16 = col.astype(jnp.int16)
    q = q0 + i * rows_per_block + jax.lax.broadcasted_iota(jnp.int32, x.shape, 0)
    vm = jnp.where(col > q, jnp.float32(-1e9), x)

    if w == k:
        _, topi = _topk_sort(vm, col16, w, k)
        idx_ref[0] = topi.astype(jnp.int32)
        colf = jax.lax.broadcasted_iota(
            jnp.int32, (rows_per_block, seq_k), 1
        )
        mask_ref[0] = (colf < k).astype(jnp.int8)
        return

    topv, topi16 = _topk_sort(vm, col16, w, k)
    topi = topi16.astype(jnp.int32)
    idx_ref[0] = topi
    thr = topv[:, k - 1 : k]
    tie_max = jnp.max(
        jnp.where(topv == thr, topi, jnp.int32(-1)), axis=1, keepdims=True
    )
    sel = (vm > thr) | ((vm == thr) & (col <= tie_max))
    mask_ref[0, :, :w] = sel.astype(jnp.int8)
    if w < seq_k:
        mask_ref[0, :, w:] = jnp.zeros(
            (rows_per_block, seq_k - w), jnp.int8
        )


def _run_class(x, q0, q1, w, k, rows_per_block, interpret=False):
    B, Sq, Sk = x.shape
    nrows = q1 - q0
    body = functools.partial(
        _class_body, rows_per_block=rows_per_block, q0=q0, w=w, seq_k=Sk, k=k
    )
    return pl.pallas_call(
        body,
        grid=(B, nrows // rows_per_block),
        in_specs=[
            pl.BlockSpec(
                (1, rows_per_block, w),
                lambda b, i, q0=q0, R=rows_per_block: (b, q0 // R + i, 0),
            )
        ],
        out_specs=[
            pl.BlockSpec((1, rows_per_block, Sk), lambda b, i: (b, i, 0)),
            pl.BlockSpec((1, rows_per_block, k), lambda b, i: (b, i, 0)),
        ],
        out_shape=[
            jax.ShapeDtypeStruct((B, nrows, Sk), jnp.int8),
            jax.ShapeDtypeStruct((B, nrows, k), jnp.int32),
        ],
        compiler_params=None
        if interpret
        else pltpu.CompilerParams(
            dimension_semantics=("parallel", "parallel")
        ),
        interpret=interpret,
    )(x)


def _run_topk(x, k, rows_per_block, interpret=False):
    B, Sq, Sk = x.shape
    masks, idxs = [], []
    for m in range(Sq // k):
        q0, q1 = m * k, (m + 1) * k
        w = min(q1, Sk)
        mm, ti = _run_class(x, q0, q1, w, k, rows_per_block, interpret)
        masks.append(mm)
        idxs.append(ti)
    mask = masks[0] if len(masks) == 1 else jnp.concatenate(masks, axis=1)
    topi = idxs[0] if len(idxs) == 1 else jnp.concatenate(idxs, axis=1)
    return mask, topi


def kernel(index_scores):
    B, Sq, Sk = index_scores.shape
    k = min(512, Sk)
    mask8, topi = _run_topk(index_scores, k, 64)
    mask = mask8 != 0
    sparsity = jnp.float32(1.0) - jnp.float32(k) / jnp.float32(Sk)
    return (mask, topi, sparsity)


# speedup vs baseline: 1.8492x; 1.0674x over previous
import functools

import jax
import jax.numpy as jnp
from jax.experimental import pallas as pl
from jax.experimental.pallas import tpu as pltpu


def _stage(v, idx, col, j, desc):
    bit = (col & j) != 0
    pv = jnp.where(bit, jnp.roll(v, j, axis=-1), jnp.roll(v, -j, axis=-1))
    pi = jnp.where(bit, jnp.roll(idx, j, axis=-1), jnp.roll(idx, -j, axis=-1))
    keep_first = desc ^ bit
    self_wins = (v > pv) | ((v == pv) & (idx < pi))
    sel = self_wins ^ keep_first
    return jnp.where(sel, pv, v), jnp.where(sel, pi, idx)


def _topk_sort(v, idx, w, k):
    col = jax.lax.broadcasted_iota(jnp.int32, v.shape, v.ndim - 1)
    kk = 2
    while kk <= k:
        desc = (col & kk) == 0
        j = kk // 2
        while j >= 1:
            v, idx = _stage(v, idx, col, j, desc)
            j //= 2
        kk *= 2
    c = w // k
    while c > 1:
        npairs = c // 2
        head_w = 2 * npairs * k
        hv, hi = v[..., :head_w], idx[..., :head_w]
        hcol = jax.lax.broadcasted_iota(jnp.int32, hv.shape, hv.ndim - 1)
        hv, hi = _stage(hv, hi, hcol, k, hcol >= 0)
        keep_v = [hv[..., p * 2 * k : p * 2 * k + k] for p in range(npairs)]
        keep_i = [hi[..., p * 2 * k : p * 2 * k + k] for p in range(npairs)]
        if c % 2:
            keep_v.append(v[..., (c - 1) * k :])
            keep_i.append(idx[..., (c - 1) * k :])
        v = keep_v[0] if len(keep_v) == 1 else jnp.concatenate(keep_v, axis=-1)
        idx = keep_i[0] if len(keep_i) == 1 else jnp.concatenate(keep_i, axis=-1)
        c = (c + 1) // 2
        col = jax.lax.broadcasted_iota(jnp.int32, v.shape, v.ndim - 1)
        desc = (col & k) == 0
        j = k // 2
        while j >= 1:
            v, idx = _stage(v, idx, col, j, desc)
            j //= 2
    return v, idx


def _class_body(x_ref, mask_ref, idx_ref, *, rows_per_block, q0, w, seq_k, k):
    i = pl.program_id(1)
    x = x_ref[0]
    col = jax.lax.broadcasted_iota(jnp.int32, x.shape, 1)
    col16 = col.astype(jnp.int16)
    q = q0 + i * rows_per_block + jax.lax.broadcasted_iota(jnp.int32, x.shape, 0)
    vm = jnp.where(col > q, jnp.float32(-1e9), x)

    if w == k:
        _, topi = _topk_sort(vm, col16, w, k)
        idx_ref[0] = topi.astype(jnp.int32)
        colf = jax.lax.broadcasted_iota(
            jnp.int32, (rows_per_block, seq_k), 1
        )
        mask_ref[0] = (colf < k).astype(jnp.int8)
        return

    topv, topi16 = _topk_sort(vm, col16, w, k)
    topi = topi16.astype(jnp.int32)
    idx_ref[0] = topi
    thr = topv[:, k - 1 : k]
    tie_max = jnp.max(
        jnp.where(topv == thr, topi, jnp.int32(-1)), axis=1, keepdims=True
    )
    sel = (vm > thr) | ((vm == thr) & (col <= tie_max))
    mask_ref[0, :, :w] = sel.astype(jnp.int8)
    if w < seq_k:
        mask_ref[0, :, w:] = jnp.zeros(
            (rows_per_block, seq_k - w), jnp.int8
        )


def _run_class(x, q0, q1, w, k, rows_per_block, interpret=False):
    B, Sq, Sk = x.shape
    nrows = q1 - q0
    body = functools.partial(
        _class_body, rows_per_block=rows_per_block, q0=q0, w=w, seq_k=Sk, k=k
    )
    return pl.pallas_call(
        body,
        grid=(B, nrows // rows_per_block),
        in_specs=[
            pl.BlockSpec(
                (1, rows_per_block, w),
                lambda b, i, q0=q0, R=rows_per_block: (b, q0 // R + i, 0),
            )
        ],
        out_specs=[
            pl.BlockSpec((1, rows_per_block, Sk), lambda b, i: (b, i, 0)),
            pl.BlockSpec((1, rows_per_block, k), lambda b, i: (b, i, 0)),
        ],
        out_shape=[
            jax.ShapeDtypeStruct((B, nrows, Sk), jnp.int8),
            jax.ShapeDtypeStruct((B, nrows, k), jnp.int32),
        ],
        compiler_params=None
        if interpret
        else pltpu.CompilerParams(
            dimension_semantics=("parallel", "parallel")
        ),
        interpret=interpret,
    )(x)


def _run_topk(x, k, rows_per_block, interpret=False):
    B, Sq, Sk = x.shape
    masks, idxs = [], []
    for m in range(Sq // k):
        q0, q1 = m * k, (m + 1) * k
        w = min(q1, Sk)
        mm, ti = _run_class(x, q0, q1, w, k, rows_per_block, interpret)
        masks.append(mm)
        idxs.append(ti)
    mask = masks[0] if len(masks) == 1 else jnp.concatenate(masks, axis=1)
    topi = idxs[0] if len(idxs) == 1 else jnp.concatenate(idxs, axis=1)
    return mask, topi


def kernel(index_scores):
    B, Sq, Sk = index_scores.shape
    k = min(512, Sk)
    mask8, topi = _run_topk(index_scores, k, 128)
    mask = mask8 != 0
    sparsity = jnp.float32(1.0) - jnp.float32(k) / jnp.float32(Sk)
    return (mask, topi, sparsity)
